# baseline (speedup 1.0000x reference)
import functools

import jax
import jax.numpy as jnp
from jax import lax
from jax.experimental import pallas as pl
from jax.experimental.pallas import tpu as pltpu
from jax.experimental.pallas import tpu_sc as plsc


def _build_kernel(N, D, n_chunks, C, num_cores, nw):
    mesh = plsc.VectorSubcoreMesh(core_axis_name="c", subcore_axis_name="s")

    @functools.partial(
        pl.kernel,
        mesh=mesh,
        out_type=jax.ShapeDtypeStruct((N, D), jnp.float32),
        scratch_types=[
            pltpu.VMEM((n_chunks, C), jnp.int32),
            pltpu.VMEM((3, C, D), jnp.float32),
            pltpu.SemaphoreType.DMA,
            pltpu.SemaphoreType.DMA,
            pltpu.SemaphoreType.DMA,
        ],
    )
    def emb_kernel(
        ids_hbm, tab_hbm, out_hbm, idx_v, rows_v, gsem0, gsem1, gsem2
    ):
        wid = lax.axis_index("s") * num_cores + lax.axis_index("c")
        base = wid * C
        stride = nw * C

        pltpu.sync_copy(ids_hbm.at[wid], idx_v)

        gsems = (gsem0, gsem1, gsem2)

        def gather(ci, b):
            return pltpu.make_async_copy(
                tab_hbm.at[idx_v.at[ci]], rows_v.at[b], gsems[b]
            )

        gather(0, 0).start()
        gather(1, 1).start()

        def body(i):
            for b in range(3):
                ci = i + b
                gather(ci + 2, (b + 2) % 3).start()
                gather(ci, b).wait()
                pltpu.sync_copy(
                    rows_v.at[b], out_hbm.at[pl.ds(base + ci * stride, C)]
                )

        pl.loop(0, n_chunks - 2, step=3)(body)

        for ci in (n_chunks - 2, n_chunks - 1):
            b = ci % 3
            gather(ci, b).wait()
            pltpu.sync_copy(rows_v.at[b], out_hbm.at[pl.ds(base + ci * stride, C)])

    return emb_kernel


def kernel(input_ids, embed_table):
    B, S = input_ids.shape
    V, D = embed_table.shape
    N = B * S

    info = plsc.get_sparse_core_info()
    NW = info.num_cores * info.num_subcores
    assert N % NW == 0
    n_per_w = N // NW
    C = 32
    assert n_per_w % C == 0
    n_chunks = n_per_w // C
    assert (n_chunks - 2) % 3 == 0

    ids = (
        input_ids.reshape(n_chunks, NW, C)
        .transpose(1, 0, 2)
        .astype(jnp.int32)
    )
    emb_kernel = _build_kernel(N, D, n_chunks, C, info.num_cores, NW)
    out = emb_kernel(ids, embed_table)
    return out.reshape(B, S, D)
